# baseline (speedup 1.0000x reference)
import jax
import jax.numpy as jnp
from jax.experimental import pallas as pl
from jax.experimental.pallas import tpu as pltpu

_BM = 200
_NBUF = 6
_SPLITS = ((0, 96), (96, 104))


def _fused_kernel(x_ref, Wgc_ref, adj_ref, t_ref, bgc_ref, W00_ref,
                  b00_ref, W10_ref, b10_ref, w01_ref, b01_ref, w11_ref,
                  b11_ref, wpp_ref, bpp_ref,
                  rep_ref, y_ref, p1_ref,
                  sup_ref, xbuf_ref, tbuf_ref, buf_ref, repbuf_ref,
                  y0scr_ref, y1scr_ref, pscr_ref,
                  sem_ref, xsem_ref, tsem_ref, repsem_ref):
    n = adj_ref.shape[0]
    nsteps = n // _BM

    def _start(step, b):
        for j, (o, sz) in enumerate(_SPLITS):
            pltpu.make_async_copy(
                adj_ref.at[pl.ds(step * _BM + o, sz), :],
                buf_ref.at[b, pl.ds(o, sz), :], sem_ref.at[b, j]).start()

    def _wait(step, b):
        for j, (o, sz) in enumerate(_SPLITS):
            pltpu.make_async_copy(
                adj_ref.at[pl.ds(step * _BM + o, sz), :],
                buf_ref.at[b, pl.ds(o, sz), :], sem_ref.at[b, j]).wait()

    for b in range(_NBUF):
        _start(b, b)

    tcopy = pltpu.make_async_copy(t_ref, tbuf_ref, tsem_ref)
    tcopy.start()
    xcopy = pltpu.make_async_copy(x_ref, xbuf_ref, xsem_ref)
    xcopy.start()
    xcopy.wait()
    sup_ref[...] = jnp.dot(xbuf_ref[...], Wgc_ref[...],
                           preferred_element_type=jnp.float32
                           ).astype(jnp.bfloat16)

    def body(s, carry):
        b = jax.lax.rem(s, _NBUF)
        row = s * _BM
        _wait(s, b)
        acc = jnp.dot(buf_ref[b].astype(jnp.bfloat16), sup_ref[...],
                      preferred_element_type=jnp.float32)
        rep = jnp.maximum(acc + bgc_ref[...], 0.0)
        rb = jax.lax.rem(s, 2)

        @pl.when(s >= 2)
        def _():
            pltpu.make_async_copy(
                repbuf_ref.at[rb],
                rep_ref.at[pl.ds((s - 2) * _BM, _BM), :],
                repsem_ref.at[rb]).wait()

        repbuf_ref[rb] = rep
        pltpu.make_async_copy(repbuf_ref.at[rb],
                              rep_ref.at[pl.ds(row, _BM), :],
                              repsem_ref.at[rb]).start()

        y00 = jnp.maximum(
            jnp.dot(rep, W00_ref[...], preferred_element_type=jnp.float32)
            + b00_ref[...], 0.0)
        y10 = jnp.maximum(
            jnp.dot(rep, W10_ref[...], preferred_element_type=jnp.float32)
            + b10_ref[...], 0.0)
        y0 = jnp.dot(y00, w01_ref[...],
                     preferred_element_type=jnp.float32) + b01_ref[...]
        y1 = jnp.dot(y10, w11_ref[...],
                     preferred_element_type=jnp.float32) + b11_ref[...]
        pp = jnp.dot(rep, wpp_ref[...],
                     preferred_element_type=jnp.float32) + bpp_ref[...]
        y0scr_ref[pl.ds(s, 1), :] = y0.reshape(1, _BM)
        y1scr_ref[pl.ds(s, 1), :] = y1.reshape(1, _BM)
        pscr_ref[pl.ds(s, 1), :] = jax.nn.sigmoid(pp.reshape(1, _BM))

        nxt = s + _NBUF

        @pl.when(nxt < nsteps)
        def _():
            _start(nxt, b)
        return carry

    jax.lax.fori_loop(0, nsteps, body, 0)

    tcopy.wait()
    y0flat = jnp.concatenate(
        [y0scr_ref[pl.ds(k, 1), :] for k in range(nsteps)], axis=1
    ).reshape(n)
    y1flat = jnp.concatenate(
        [y1scr_ref[pl.ds(k, 1), :] for k in range(nsteps)], axis=1
    ).reshape(n)
    y_ref[...] = jnp.where(tbuf_ref[...] > 0, y1flat, y0flat)
    p1_ref[...] = jnp.concatenate(
        [pscr_ref[pl.ds(k, 1), :] for k in range(nsteps)], axis=1
    ).reshape(n)

    for i in range(2):
        sfin = nsteps - 2 + i
        pltpu.make_async_copy(
            repbuf_ref.at[sfin % 2],
            rep_ref.at[pl.ds(sfin * _BM, _BM), :],
            repsem_ref.at[sfin % 2]).wait()


def kernel(x, adj, t, W_gc, b_gc, W00, b00, W10, b10, w01, b01, w11,
           b11, wpp, bpp):
    N, F = x.shape
    H = W_gc.shape[1]
    nsteps = N // _BM

    t2 = t.astype(jnp.int32)
    bgc2 = b_gc.reshape(1, H)
    b002 = b00.reshape(1, H)
    b102 = b10.reshape(1, H)
    b012 = b01.reshape(1, 1)
    b112 = b11.reshape(1, 1)
    bpp2 = bpp.reshape(1, 1)

    vmem = pl.BlockSpec(memory_space=pltpu.VMEM)
    hbm = pl.BlockSpec(memory_space=pl.ANY)
    rep, y, p1 = pl.pallas_call(
        _fused_kernel,
        in_specs=[hbm, vmem, hbm, hbm,
                  vmem, vmem, vmem, vmem, vmem, vmem, vmem, vmem,
                  vmem, vmem, vmem],
        out_specs=[hbm, vmem, vmem],
        out_shape=[
            jax.ShapeDtypeStruct((N, H), jnp.float32),
            jax.ShapeDtypeStruct((N,), jnp.float32),
            jax.ShapeDtypeStruct((N,), jnp.float32),
        ],
        scratch_shapes=[
            pltpu.VMEM((N, H), jnp.bfloat16),
            pltpu.VMEM((N, F), jnp.float32),
            pltpu.VMEM((N,), jnp.int32),
            pltpu.VMEM((_NBUF, _BM, N), jnp.float32),
            pltpu.VMEM((2, _BM, H), jnp.float32),
            pltpu.VMEM((nsteps, _BM), jnp.float32),
            pltpu.VMEM((nsteps, _BM), jnp.float32),
            pltpu.VMEM((nsteps, _BM), jnp.float32),
            pltpu.SemaphoreType.DMA((_NBUF, len(_SPLITS))),
            pltpu.SemaphoreType.DMA,
            pltpu.SemaphoreType.DMA,
            pltpu.SemaphoreType.DMA((2,)),
        ],
    )(x, W_gc, adj, t2, bgc2, W00, b002, W10, b102,
      w01, b012, w11, b112, wpp, bpp2)

    return y, rep, p1


# speedup vs baseline: 1.0199x; 1.0199x over previous
import jax
import jax.numpy as jnp
from jax.experimental import pallas as pl
from jax.experimental.pallas import tpu as pltpu

_BM = 200
_NBUF = 5
_SPLITS = ((0, 96), (96, 104))


def _fused_kernel(x_ref, Wgc_ref, adj_ref, t_ref, bgc_ref, W00_ref,
                  b00_ref, W10_ref, b10_ref, w01_ref, b01_ref, w11_ref,
                  b11_ref, wpp_ref, bpp_ref,
                  rep_ref, y_ref, p1_ref,
                  sup_ref, xbuf_ref, tbuf_ref, buf_ref, repbuf_ref,
                  y0scr_ref, y1scr_ref, pscr_ref,
                  sem_ref, xsem_ref, tsem_ref, repsem_ref):
    n = adj_ref.shape[0]
    nsteps = n // _BM

    def _start(step, b):
        for j, (o, sz) in enumerate(_SPLITS):
            pltpu.make_async_copy(
                adj_ref.at[pl.ds(step * _BM + o, sz), :],
                buf_ref.at[b, pl.ds(o, sz), :], sem_ref.at[b, j]).start()

    def _wait(step, b):
        for j, (o, sz) in enumerate(_SPLITS):
            pltpu.make_async_copy(
                adj_ref.at[pl.ds(step * _BM + o, sz), :],
                buf_ref.at[b, pl.ds(o, sz), :], sem_ref.at[b, j]).wait()

    xcopy = pltpu.make_async_copy(x_ref, xbuf_ref, xsem_ref)
    xcopy.start()
    tcopy = pltpu.make_async_copy(t_ref, tbuf_ref, tsem_ref)
    tcopy.start()

    for b in range(_NBUF):
        _start(b, b)

    xcopy.wait()
    sup_ref[...] = jnp.dot(xbuf_ref[...], Wgc_ref[...],
                           preferred_element_type=jnp.float32
                           ).astype(jnp.bfloat16)

    def body(s, carry):
        b = jax.lax.rem(s, _NBUF)
        row = s * _BM
        _wait(s, b)
        acc = jnp.dot(buf_ref[b].astype(jnp.bfloat16), sup_ref[...],
                      preferred_element_type=jnp.float32)
        rep = jnp.maximum(acc + bgc_ref[...], 0.0)
        rb = jax.lax.rem(s, 2)

        @pl.when(s >= 2)
        def _():
            pltpu.make_async_copy(
                repbuf_ref.at[rb],
                rep_ref.at[pl.ds((s - 2) * _BM, _BM), :],
                repsem_ref.at[rb]).wait()

        repbuf_ref[rb] = rep
        pltpu.make_async_copy(repbuf_ref.at[rb],
                              rep_ref.at[pl.ds(row, _BM), :],
                              repsem_ref.at[rb]).start()

        y00 = jnp.maximum(
            jnp.dot(rep, W00_ref[...], preferred_element_type=jnp.float32)
            + b00_ref[...], 0.0)
        y10 = jnp.maximum(
            jnp.dot(rep, W10_ref[...], preferred_element_type=jnp.float32)
            + b10_ref[...], 0.0)
        y0 = jnp.dot(y00, w01_ref[...],
                     preferred_element_type=jnp.float32) + b01_ref[...]
        y1 = jnp.dot(y10, w11_ref[...],
                     preferred_element_type=jnp.float32) + b11_ref[...]
        pp = jnp.dot(rep, wpp_ref[...],
                     preferred_element_type=jnp.float32) + bpp_ref[...]
        y0scr_ref[pl.ds(s, 1), :] = y0.reshape(1, _BM)
        y1scr_ref[pl.ds(s, 1), :] = y1.reshape(1, _BM)
        pscr_ref[pl.ds(s, 1), :] = jax.nn.sigmoid(pp.reshape(1, _BM))

        nxt = s + _NBUF

        @pl.when(nxt < nsteps)
        def _():
            _start(nxt, b)
        return carry

    jax.lax.fori_loop(0, nsteps, body, 0)

    tcopy.wait()
    y0flat = jnp.concatenate(
        [y0scr_ref[pl.ds(k, 1), :] for k in range(nsteps)], axis=1
    ).reshape(n)
    y1flat = jnp.concatenate(
        [y1scr_ref[pl.ds(k, 1), :] for k in range(nsteps)], axis=1
    ).reshape(n)
    y_ref[...] = jnp.where(tbuf_ref[...] > 0, y1flat, y0flat)
    p1_ref[...] = jnp.concatenate(
        [pscr_ref[pl.ds(k, 1), :] for k in range(nsteps)], axis=1
    ).reshape(n)

    for i in range(2):
        sfin = nsteps - 2 + i
        pltpu.make_async_copy(
            repbuf_ref.at[sfin % 2],
            rep_ref.at[pl.ds(sfin * _BM, _BM), :],
            repsem_ref.at[sfin % 2]).wait()


def kernel(x, adj, t, W_gc, b_gc, W00, b00, W10, b10, w01, b01, w11,
           b11, wpp, bpp):
    N, F = x.shape
    H = W_gc.shape[1]
    nsteps = N // _BM

    t2 = t.astype(jnp.int32)
    bgc2 = b_gc.reshape(1, H)
    b002 = b00.reshape(1, H)
    b102 = b10.reshape(1, H)
    b012 = b01.reshape(1, 1)
    b112 = b11.reshape(1, 1)
    bpp2 = bpp.reshape(1, 1)

    vmem = pl.BlockSpec(memory_space=pltpu.VMEM)
    hbm = pl.BlockSpec(memory_space=pl.ANY)
    rep, y, p1 = pl.pallas_call(
        _fused_kernel,
        in_specs=[hbm, vmem, hbm, hbm,
                  vmem, vmem, vmem, vmem, vmem, vmem, vmem, vmem,
                  vmem, vmem, vmem],
        out_specs=[hbm, vmem, vmem],
        out_shape=[
            jax.ShapeDtypeStruct((N, H), jnp.float32),
            jax.ShapeDtypeStruct((N,), jnp.float32),
            jax.ShapeDtypeStruct((N,), jnp.float32),
        ],
        scratch_shapes=[
            pltpu.VMEM((N, H), jnp.bfloat16),
            pltpu.VMEM((N, F), jnp.float32),
            pltpu.VMEM((N,), jnp.int32),
            pltpu.VMEM((_NBUF, _BM, N), jnp.float32),
            pltpu.VMEM((2, _BM, H), jnp.float32),
            pltpu.VMEM((nsteps, _BM), jnp.float32),
            pltpu.VMEM((nsteps, _BM), jnp.float32),
            pltpu.VMEM((nsteps, _BM), jnp.float32),
            pltpu.SemaphoreType.DMA((_NBUF, len(_SPLITS))),
            pltpu.SemaphoreType.DMA,
            pltpu.SemaphoreType.DMA,
            pltpu.SemaphoreType.DMA((2,)),
        ],
    )(x, W_gc, adj, t2, bgc2, W00, b002, W10, b102,
      w01, b012, w11, b112, wpp, bpp2)

    return y, rep, p1
